# baseline (device time: 35052 ns/iter reference)
import jax
import jax.numpy as jnp
from jax import lax
from jax.experimental import pallas as pl
from jax.experimental.pallas import tpu as pltpu

N_DEV = 4
B_PER = 2
SQ = 256
SKV = 256
NH = 16
H_PER = 4
DH = 64
D_MODEL = 512
HD = H_PER * DH
KW = NH * DH
BLK = 64
SCALE = 0.125
BF16 = jnp.bfloat16


def kernel(x, Wq, K_ext, V_ext, Wo):
    wq_b = Wq.astype(BF16)
    wo_b = Wo.astype(BF16)
    k_flat = jnp.reshape(K_ext, (N_DEV * B_PER, SKV, KW))
    v_flat = jnp.reshape(V_ext, (N_DEV * B_PER, SKV, KW))

    def body(x_ref, wq_ref, k_hbm, v_hbm, wo_ref, out_ref,
             wq_all, wo_all, x_bf, k32, v32, kb, vb,
             kv_sems, send_sems, recv_sems):
        my_pos = lax.axis_index("i")

        ck32 = pltpu.make_async_copy(
            k_hbm.at[pl.ds(my_pos * B_PER, B_PER)], k32, kv_sems.at[0])
        cv32 = pltpu.make_async_copy(
            v_hbm.at[pl.ds(my_pos * B_PER, B_PER)], v32, kv_sems.at[1])
        ck32.start()
        cv32.start()

        barrier = pltpu.get_barrier_semaphore()
        for off in range(1, N_DEV):
            pl.semaphore_signal(
                barrier, inc=1,
                device_id=((my_pos + off) % N_DEV,),
                device_id_type=pl.DeviceIdType.MESH,
            )
        pl.semaphore_wait(barrier, N_DEV - 1)

        rdmas = []
        for off in (3, 2, 1):
            dst = (my_pos + off) % N_DEV
            for t, src, buf in ((0, wq_ref, wq_all), (1, wo_ref, wo_all)):
                r = pltpu.make_async_remote_copy(
                    src_ref=src,
                    dst_ref=buf.at[my_pos],
                    send_sem=send_sems.at[t, off],
                    recv_sem=recv_sems.at[t, off],
                    device_id=(dst,),
                    device_id_type=pl.DeviceIdType.MESH,
                )
                r.start()
                rdmas.append(r)

        for b in range(B_PER):
            x_bf[b] = x_ref[b].astype(BF16)

        ck32.wait()
        cv32.wait()
        for b in range(B_PER):
            kb[b] = k32[b].astype(BF16)
            vb[b] = v32[b].astype(BF16)

        qb_i = lax.broadcasted_iota(jnp.int32, (SQ, SKV), 0) // BLK
        kb_i = lax.broadcasted_iota(jnp.int32, (SQ, SKV), 1) // BLK
        mask = kb_i <= qb_i

        def wait_weight(t, buf, j, sem_off):
            pltpu.make_async_remote_copy(
                src_ref=buf.at[j],
                dst_ref=buf.at[j],
                send_sem=send_sems.at[t, sem_off],
                recv_sem=recv_sems.at[t, sem_off],
                device_id=(j,),
                device_id_type=pl.DeviceIdType.MESH,
            ).wait_recv()

        for off in range(N_DEV):
            j = (my_pos + off) % N_DEV
            sem_off = N_DEV - off

            rows = lax.broadcasted_iota(jnp.int32, (KW, HD), 0)
            cols = lax.broadcasted_iota(jnp.int32, (KW, HD), 1)
            sel = (rows == cols + j * HD).astype(BF16)
            kg = []
            vg = []
            for b in range(B_PER):
                kg.append(jnp.dot(
                    kb[b], sel, preferred_element_type=jnp.float32
                ).astype(BF16))
                vg.append(jnp.dot(
                    vb[b], sel, preferred_element_type=jnp.float32
                ).astype(BF16))

            if off:
                wait_weight(0, wq_all, j, sem_off)
            wqj = wq_ref[...] if off == 0 else wq_all[j]

            ctxs = []
            for b in range(B_PER):
                qj = (jnp.dot(
                    x_bf[b], wqj, preferred_element_type=jnp.float32
                ) * SCALE).astype(BF16)
                for hh in range(H_PER):
                    q = qj[:, hh * DH:(hh + 1) * DH]
                    k = kg[b][:, hh * DH:(hh + 1) * DH]
                    s = lax.dot_general(
                        q, k, (((1,), (1,)), ((), ())),
                        preferred_element_type=jnp.float32,
                    )
                    e = jnp.exp(jnp.where(mask, s, -1e9).astype(BF16))
                    den = jnp.sum(e, axis=-1, keepdims=True,
                                  dtype=jnp.float32)
                    v = vg[b][:, hh * DH:(hh + 1) * DH]
                    ctx = jnp.dot(
                        e, v, preferred_element_type=jnp.float32
                    ) / den
                    ctxs.append(ctx.astype(BF16))

            if off:
                wait_weight(1, wo_all, j, sem_off)
            woj = wo_ref[...] if off == 0 else wo_all[j]
            for b in range(B_PER):
                accs = [
                    jnp.dot(
                        ctxs[b * H_PER + hh],
                        woj[hh * DH:(hh + 1) * DH, :],
                        preferred_element_type=jnp.float32,
                    )
                    for hh in range(H_PER)
                ]
                contrib = (accs[0] + accs[1]) + (accs[2] + accs[3])
                if off == 0:
                    out_ref[b] = contrib
                else:
                    out_ref[b] = out_ref[b] + contrib

        for r in rdmas:
            r.wait_send()

    return pl.pallas_call(
        body,
        out_shape=jax.ShapeDtypeStruct((B_PER, SQ, D_MODEL), jnp.float32),
        in_specs=[
            pl.BlockSpec(memory_space=pltpu.VMEM),
            pl.BlockSpec(memory_space=pltpu.VMEM),
            pl.BlockSpec(memory_space=pltpu.MemorySpace.HBM),
            pl.BlockSpec(memory_space=pltpu.MemorySpace.HBM),
            pl.BlockSpec(memory_space=pltpu.VMEM),
        ],
        out_specs=pl.BlockSpec(memory_space=pltpu.VMEM),
        scratch_shapes=[
            pltpu.VMEM((N_DEV, D_MODEL, HD), BF16),
            pltpu.VMEM((N_DEV, HD, D_MODEL), BF16),
            pltpu.VMEM((B_PER, SQ, D_MODEL), BF16),
            pltpu.VMEM((B_PER, SKV, KW), jnp.float32),
            pltpu.VMEM((B_PER, SKV, KW), jnp.float32),
            pltpu.VMEM((B_PER, SKV, KW), BF16),
            pltpu.VMEM((B_PER, SKV, KW), BF16),
            pltpu.SemaphoreType.DMA((2,)),
            pltpu.SemaphoreType.DMA((2, N_DEV)),
            pltpu.SemaphoreType.DMA((2, N_DEV)),
        ],
        compiler_params=pltpu.CompilerParams(collective_id=0),
    )(x, wq_b, k_flat, v_flat, wo_b)


# device time: 27052 ns/iter; 1.2957x vs baseline; 1.2957x over previous
import jax
import jax.numpy as jnp
from jax import lax
from jax.experimental import pallas as pl
from jax.experimental.pallas import tpu as pltpu

N_DEV = 4
B_PER = 2
SQ = 256
SKV = 256
H_PER = 4
DH = 64
D_MODEL = 512
HD = H_PER * DH
BLK = 64
SCALE = 0.125
BF16 = jnp.bfloat16


def kernel(x, Wq, K_ext, V_ext, Wo):
    my = lax.axis_index("i")
    k_loc = jnp.transpose(
        lax.dynamic_slice_in_dim(K_ext, my * B_PER, B_PER, axis=0).astype(BF16),
        (2, 0, 1, 3),
    )
    v_loc = jnp.transpose(
        lax.dynamic_slice_in_dim(V_ext, my * B_PER, B_PER, axis=0).astype(BF16),
        (2, 0, 1, 3),
    )
    wq_b = Wq.astype(BF16)
    wo_b = Wo.astype(BF16)

    def body(x_ref, wq_ref, k_ref, v_ref, wo_ref, out_ref,
             wq_all, wo_all, x_bf, send_sems, recv_sems):
        my_pos = lax.axis_index("i")

        barrier = pltpu.get_barrier_semaphore()
        for off in range(1, N_DEV):
            pl.semaphore_signal(
                barrier, inc=1,
                device_id=((my_pos + off) % N_DEV,),
                device_id_type=pl.DeviceIdType.MESH,
            )
        pl.semaphore_wait(barrier, N_DEV - 1)

        rdmas = []
        for off in (3, 2, 1):
            dst = (my_pos + off) % N_DEV
            for t, src, buf in ((0, wq_ref, wq_all), (1, wo_ref, wo_all)):
                r = pltpu.make_async_remote_copy(
                    src_ref=src,
                    dst_ref=buf.at[my_pos],
                    send_sem=send_sems.at[t, off],
                    recv_sem=recv_sems.at[t, off],
                    device_id=(dst,),
                    device_id_type=pl.DeviceIdType.MESH,
                )
                r.start()
                rdmas.append(r)

        for b in range(B_PER):
            x_bf[b] = x_ref[b].astype(BF16)

        qb_i = lax.broadcasted_iota(jnp.int32, (SQ, SKV), 0) // BLK
        kb_i = lax.broadcasted_iota(jnp.int32, (SQ, SKV), 1) // BLK
        mask = kb_i <= qb_i

        def wait_weight(t, buf, j, sem_off):
            pltpu.make_async_remote_copy(
                src_ref=buf.at[j],
                dst_ref=buf.at[j],
                send_sem=send_sems.at[t, sem_off],
                recv_sem=recv_sems.at[t, sem_off],
                device_id=(j,),
                device_id_type=pl.DeviceIdType.MESH,
            ).wait_recv()

        for off in range(N_DEV):
            j = (my_pos + off) % N_DEV
            sem_off = N_DEV - off

            if off:
                wait_weight(0, wq_all, j, sem_off)
            wqj = wq_ref[...] if off == 0 else wq_all[j]

            ctxs = []
            for b in range(B_PER):
                qj = (jnp.dot(
                    x_bf[b], wqj, preferred_element_type=jnp.float32
                ) * SCALE).astype(BF16)
                for hh in range(H_PER):
                    h = j * H_PER + hh
                    q = qj[:, hh * DH:(hh + 1) * DH]
                    s = lax.dot_general(
                        q, k_ref[h, b], (((1,), (1,)), ((), ())),
                        preferred_element_type=jnp.float32,
                    )
                    e = jnp.exp(jnp.where(mask, s, -1e9).astype(BF16))
                    den = jnp.sum(e, axis=-1, keepdims=True,
                                  dtype=jnp.float32)
                    ctx = jnp.dot(
                        e, v_ref[h, b], preferred_element_type=jnp.float32
                    ) / den
                    ctxs.append(ctx.astype(BF16))

            if off:
                wait_weight(1, wo_all, j, sem_off)
            woj = wo_ref[...] if off == 0 else wo_all[j]
            for b in range(B_PER):
                accs = [
                    jnp.dot(
                        ctxs[b * H_PER + hh],
                        woj[hh * DH:(hh + 1) * DH, :],
                        preferred_element_type=jnp.float32,
                    )
                    for hh in range(H_PER)
                ]
                contrib = (accs[0] + accs[1]) + (accs[2] + accs[3])
                if off == 0:
                    out_ref[b] = contrib
                else:
                    out_ref[b] = out_ref[b] + contrib

        for r in rdmas:
            r.wait_send()

    return pl.pallas_call(
        body,
        out_shape=jax.ShapeDtypeStruct((B_PER, SQ, D_MODEL), jnp.float32),
        in_specs=[pl.BlockSpec(memory_space=pltpu.VMEM)] * 5,
        out_specs=pl.BlockSpec(memory_space=pltpu.VMEM),
        scratch_shapes=[
            pltpu.VMEM((N_DEV, D_MODEL, HD), BF16),
            pltpu.VMEM((N_DEV, HD, D_MODEL), BF16),
            pltpu.VMEM((B_PER, SQ, D_MODEL), BF16),
            pltpu.SemaphoreType.DMA((2, N_DEV)),
            pltpu.SemaphoreType.DMA((2, N_DEV)),
        ],
        compiler_params=pltpu.CompilerParams(collective_id=0),
    )(x, wq_b, k_loc, v_loc, wo_b)
